# baseline (speedup 1.0000x reference)
import jax
import jax.numpy as jnp
from jax.experimental import pallas as pl
from jax.experimental.pallas import tpu as pltpu

_TEMP = 0.1
_BASE_TEMP = 0.07
_TOPK = 16
_G = 8
_GN = 2048
_C = 128
_RB = 256
_NRB = _GN // _RB
_SCALE = -(_TEMP / _BASE_TEMP) / (_G * _GN * 2.0)


def _fused_body(fc_ref, ft_ref, dm_ref, out_ref, fcn, ftn, lab):
    gi = pl.program_id(0)
    rb = pl.program_id(1)

    @pl.when(rb == 0)
    def _prep():
        fc = fc_ref[0]
        nc = jnp.sqrt(jnp.sum(fc * fc, axis=-1, keepdims=True))
        fcn[...] = fc / jnp.maximum(nc, 1e-12)
        ft = ft_ref[0]
        nt = jnp.sqrt(jnp.sum(ft * ft, axis=-1, keepdims=True))
        ftn[...] = ft / jnp.maximum(nt, 1e-12)
        dm = dm_ref[0]
        mx = jnp.max(dm, axis=1, keepdims=True)
        sidx = jax.lax.broadcasted_iota(jnp.int32, dm.shape, 1)
        cand = jnp.where(dm == mx, sidx, dm.shape[1])
        lab[...] = jnp.min(cand, axis=1).reshape(1, _GN).astype(jnp.float32)

    r0 = rb * _RB
    fcn_all = fcn[...]
    ftn_all = ftn[...]
    fcb = fcn[pl.ds(r0, _RB), :]
    sim = jax.lax.dot_general(
        fcb, fcn_all, (((1,), (1,)), ((), ())),
        preferred_element_type=jnp.float32)

    neg = jnp.float32(-jnp.inf)
    t = jnp.full((_RB, 1), jnp.inf, jnp.float32)
    thr = jnp.zeros((_RB, 1), jnp.float32)
    done = jnp.zeros((_RB, 1), jnp.bool_)
    for _ in range(_TOPK):
        m = jnp.max(jnp.where(sim < t, sim, neg), axis=1, keepdims=True)
        c = jnp.sum(jnp.where(sim >= m, 1.0, 0.0), axis=1, keepdims=True)
        reach = c >= _TOPK
        hit = jnp.logical_and(jnp.logical_not(done), reach)
        thr = jnp.where(hit, m, thr)
        done = jnp.logical_or(done, reach)
        t = jnp.where(done, t, m)

    ftb = ftn[pl.ds(r0, _RB), :]
    logits = jax.lax.dot_general(
        ftb, ftn_all, (((1,), (1,)), ((), ())),
        preferred_element_type=jnp.float32) / _TEMP

    col = jax.lax.broadcasted_iota(jnp.int32, (_RB, _GN), 1)
    row = jax.lax.broadcasted_iota(jnp.int32, (_RB, _GN), 0) + r0
    offd = jnp.where(col != row, 1.0, 0.0)

    el = jnp.exp(logits) * offd
    denom = jnp.sum(el, axis=1, keepdims=True)
    log_prob = logits - jnp.log(denom)

    pos = jnp.where(sim >= thr, offd, 0.0)
    pos_sum = jnp.sum(pos * log_prob, axis=1, keepdims=True)
    pos_cnt = jnp.sum(pos, axis=1, keepdims=True)

    labr = lab[:, pl.ds(r0, _RB)].reshape(_RB, 1)
    labc = lab[...]
    semi = jnp.where(labr == labc, offd, 0.0)
    semi_sum = jnp.sum(semi * log_prob, axis=1, keepdims=True)
    semi_cnt = jnp.sum(semi, axis=1, keepdims=True)

    row_loss = (pos_sum / (pos_cnt + 1e-8)
                + semi_sum / (semi_cnt + 1e-8))
    contrib = (jnp.sum(row_loss) * _SCALE).reshape(1, 1)

    @pl.when(rb == 0)
    def _init():
        out_ref[...] = jnp.zeros_like(out_ref)

    out_ref[...] += jnp.broadcast_to(contrib, out_ref.shape)


def kernel(feat_trainable, feat_criterion, dec_masks):
    ft = feat_trainable.reshape(_G, _GN, _C)
    fc = feat_criterion.reshape(_G, _GN, _C)
    dm = dec_masks.reshape(_G, 2, 16, 1024)

    out = pl.pallas_call(
        _fused_body,
        grid=(_G, _NRB),
        in_specs=[
            pl.BlockSpec((1, _GN, _C), lambda gi, rb: (gi, 0, 0)),
            pl.BlockSpec((1, _GN, _C), lambda gi, rb: (gi, 0, 0)),
            pl.BlockSpec((1, 2, 16, 1024), lambda gi, rb: (gi, 0, 0, 0)),
        ],
        out_specs=pl.BlockSpec((1, 1, 128), lambda gi, rb: (gi, 0, 0)),
        out_shape=jax.ShapeDtypeStruct((_G, 1, 128), jnp.float32),
        compiler_params=pltpu.CompilerParams(
            dimension_semantics=("parallel", "arbitrary")),
        scratch_shapes=[
            pltpu.VMEM((_GN, _C), jnp.float32),
            pltpu.VMEM((_GN, _C), jnp.float32),
            pltpu.VMEM((1, _GN), jnp.float32),
        ],
    )(fc, ft, dm)
    return jnp.sum(out[:, 0, 0])


# speedup vs baseline: 1.7192x; 1.7192x over previous
import jax
import jax.numpy as jnp
from jax.experimental import pallas as pl
from jax.experimental.pallas import tpu as pltpu

_TEMP = 0.1
_BASE_TEMP = 0.07
_TOPK = 16
_G = 8
_GN = 2048
_C = 128
_RB = 256
_NRB = _GN // _RB
_SCALE = -(_TEMP / _BASE_TEMP) / (_G * _GN * 2.0)


def _fused_body(fc_ref, ft_ref, dm_ref, out_ref, fcn, ftn, lab):
    gi = pl.program_id(0)
    rb = pl.program_id(1)

    @pl.when(rb == 0)
    def _prep():
        fc = fc_ref[0]
        nc = jnp.sqrt(jnp.sum(fc * fc, axis=-1, keepdims=True))
        fcn[...] = fc / jnp.maximum(nc, 1e-12)
        ft = ft_ref[0]
        nt = jnp.sqrt(jnp.sum(ft * ft, axis=-1, keepdims=True))
        ftn[...] = ft / jnp.maximum(nt, 1e-12)
        dm = dm_ref[0]
        mx = jnp.max(dm, axis=1, keepdims=True)
        sidx = jax.lax.broadcasted_iota(jnp.int32, dm.shape, 1)
        cand = jnp.where(dm == mx, sidx, dm.shape[1])
        lab[...] = jnp.min(cand, axis=1).reshape(1, _GN).astype(jnp.float32)

    r0 = rb * _RB
    fcn_all = fcn[...]
    ftn_all = ftn[...]
    fcb = fcn[pl.ds(r0, _RB), :]
    sim = jax.lax.dot_general(
        fcb, fcn_all, (((1,), (1,)), ((), ())),
        preferred_element_type=jnp.float32)

    neg = jnp.float32(-jnp.inf)
    thr = jnp.max(sim, axis=1, keepdims=True)
    for _ in range(_TOPK - 1):
        thr = jnp.max(jnp.where(sim < thr, sim, neg), axis=1, keepdims=True)

    ftb = ftn[pl.ds(r0, _RB), :]
    logits = jax.lax.dot_general(
        ftb, ftn_all, (((1,), (1,)), ((), ())),
        preferred_element_type=jnp.float32) / _TEMP

    col = jax.lax.broadcasted_iota(jnp.int32, (_RB, _GN), 1)
    row = jax.lax.broadcasted_iota(jnp.int32, (_RB, _GN), 0) + r0
    offd = jnp.where(col != row, 1.0, 0.0)

    el = jnp.exp(logits) * offd
    denom = jnp.sum(el, axis=1, keepdims=True)
    log_prob = logits - jnp.log(denom)

    pos = jnp.where(sim >= thr, offd, 0.0)
    pos_sum = jnp.sum(pos * log_prob, axis=1, keepdims=True)
    pos_cnt = jnp.sum(pos, axis=1, keepdims=True)

    labr = lab[:, pl.ds(r0, _RB)].reshape(_RB, 1)
    labc = lab[...]
    semi = jnp.where(labr == labc, offd, 0.0)
    semi_sum = jnp.sum(semi * log_prob, axis=1, keepdims=True)
    semi_cnt = jnp.sum(semi, axis=1, keepdims=True)

    row_loss = (pos_sum / (pos_cnt + 1e-8)
                + semi_sum / (semi_cnt + 1e-8))
    contrib = (jnp.sum(row_loss) * _SCALE).reshape(1, 1)

    @pl.when(rb == 0)
    def _init():
        out_ref[...] = jnp.zeros_like(out_ref)

    out_ref[...] += jnp.broadcast_to(contrib, out_ref.shape)


def kernel(feat_trainable, feat_criterion, dec_masks):
    ft = feat_trainable.reshape(_G, _GN, _C)
    fc = feat_criterion.reshape(_G, _GN, _C)
    dm = dec_masks.reshape(_G, 2, 16, 1024)

    out = pl.pallas_call(
        _fused_body,
        grid=(_G, _NRB),
        in_specs=[
            pl.BlockSpec((1, _GN, _C), lambda gi, rb: (gi, 0, 0)),
            pl.BlockSpec((1, _GN, _C), lambda gi, rb: (gi, 0, 0)),
            pl.BlockSpec((1, 2, 16, 1024), lambda gi, rb: (gi, 0, 0, 0)),
        ],
        out_specs=pl.BlockSpec((1, 1, 128), lambda gi, rb: (gi, 0, 0)),
        out_shape=jax.ShapeDtypeStruct((_G, 1, 128), jnp.float32),
        compiler_params=pltpu.CompilerParams(
            dimension_semantics=("parallel", "arbitrary")),
        scratch_shapes=[
            pltpu.VMEM((_GN, _C), jnp.float32),
            pltpu.VMEM((_GN, _C), jnp.float32),
            pltpu.VMEM((1, _GN), jnp.float32),
        ],
    )(fc, ft, dm)
    return jnp.sum(out[:, 0, 0])


# speedup vs baseline: 2.1714x; 1.2630x over previous
import jax
import jax.numpy as jnp
from jax.experimental import pallas as pl
from jax.experimental.pallas import tpu as pltpu

_TEMP = 0.1
_BASE_TEMP = 0.07
_TOPK = 16
_G = 8
_GN = 2048
_C = 128
_RB = 256
_NRB = _GN // _RB
_SCALE = -(_TEMP / _BASE_TEMP) / (_G * _GN * 2.0)


def _fused_body(fc_ref, ft_ref, dm_ref, out_ref, fcn, ftn, lab, thr_s):
    gi = pl.program_id(0)
    rb = pl.program_id(1)

    @pl.when(rb == 0)
    def _prep():
        fc = fc_ref[0]
        nc = jnp.sqrt(jnp.sum(fc * fc, axis=-1, keepdims=True))
        fcn[...] = fc / jnp.maximum(nc, 1e-12)
        ft = ft_ref[0]
        nt = jnp.sqrt(jnp.sum(ft * ft, axis=-1, keepdims=True))
        ftn[...] = ft / jnp.maximum(nt, 1e-12)
        dm = dm_ref[0]
        mx = jnp.max(dm, axis=1, keepdims=True)
        sidx = jax.lax.broadcasted_iota(jnp.int32, dm.shape, 1)
        cand = jnp.where(dm == mx, sidx, dm.shape[1])
        lab[...] = jnp.min(cand, axis=1).reshape(1, _GN).astype(jnp.float32)

    r0 = rb * _RB
    fcn_all = fcn[...]
    ftn_all = ftn[...]
    fcb = fcn[pl.ds(r0, _RB), :]
    sim = jax.lax.dot_general(
        fcb, fcn_all, (((1,), (1,)), ((), ())),
        preferred_element_type=jnp.float32)

    neg = jnp.float32(-jnp.inf)
    m1 = jnp.full((_RB, 128), neg, jnp.float32)
    m2 = jnp.full((_RB, 128), neg, jnp.float32)
    m3 = jnp.full((_RB, 128), neg, jnp.float32)
    m4 = jnp.full((_RB, 128), neg, jnp.float32)
    for j in range(_GN // 128):
        v = sim[:, j * 128:(j + 1) * 128]
        hi = jnp.maximum(m1, v); v = jnp.minimum(m1, v); m1 = hi
        hi = jnp.maximum(m2, v); v = jnp.minimum(m2, v); m2 = hi
        hi = jnp.maximum(m3, v); v = jnp.minimum(m3, v); m3 = hi
        m4 = jnp.maximum(m4, v)
    cand = jnp.concatenate([m1, m2, m3, m4], axis=1)
    t = jnp.max(m1, axis=1, keepdims=True)
    for _ in range(_TOPK - 1):
        t = jnp.max(jnp.where(cand < t, cand, neg), axis=1, keepdims=True)
    cnt_gt = jnp.sum(jnp.where(sim > t, 1.0, 0.0), axis=1, keepdims=True)
    all_ok = jnp.all(cnt_gt < _TOPK)
    thr_s[...] = jnp.broadcast_to(t, (_RB, 128))

    @pl.when(jnp.logical_not(all_ok))
    def _exact_topk():
        tt = jnp.full((_RB, 1), jnp.inf, jnp.float32)
        th = jnp.zeros((_RB, 1), jnp.float32)
        done = jnp.zeros((_RB, 1), jnp.bool_)
        for _ in range(_TOPK):
            m = jnp.max(jnp.where(sim < tt, sim, neg), axis=1, keepdims=True)
            c = jnp.sum(jnp.where(sim >= m, 1.0, 0.0), axis=1, keepdims=True)
            reach = c >= _TOPK
            hit = jnp.logical_and(jnp.logical_not(done), reach)
            th = jnp.where(hit, m, th)
            done = jnp.logical_or(done, reach)
            tt = jnp.where(done, tt, m)
        thr_s[...] = jnp.broadcast_to(th, (_RB, 128))

    thr = thr_s[:, 0:1]

    ftb = ftn[pl.ds(r0, _RB), :]
    logits = jax.lax.dot_general(
        ftb, ftn_all, (((1,), (1,)), ((), ())),
        preferred_element_type=jnp.float32) / _TEMP

    col = jax.lax.broadcasted_iota(jnp.int32, (_RB, _GN), 1)
    row = jax.lax.broadcasted_iota(jnp.int32, (_RB, _GN), 0) + r0
    offd = jnp.where(col != row, 1.0, 0.0)

    dsq = jnp.sum(ftb * ftb, axis=1, keepdims=True) / _TEMP
    denom = jnp.sum(jnp.exp(logits), axis=1, keepdims=True) - jnp.exp(dsq)
    logden = jnp.log(denom)

    pos = jnp.where(sim >= thr, offd, 0.0)
    pos_sum = jnp.sum(pos * logits, axis=1, keepdims=True)
    pos_cnt = jnp.sum(pos, axis=1, keepdims=True)

    labr = lab[:, pl.ds(r0, _RB)].reshape(_RB, 1)
    labc = lab[...]
    semi = jnp.where(labr == labc, offd, 0.0)
    semi_sum = jnp.sum(semi * logits, axis=1, keepdims=True)
    semi_cnt = jnp.sum(semi, axis=1, keepdims=True)

    row_loss = ((pos_sum - pos_cnt * logden) / (pos_cnt + 1e-8)
                + (semi_sum - semi_cnt * logden) / (semi_cnt + 1e-8))
    contrib = (jnp.sum(row_loss) * _SCALE).reshape(1, 1)

    @pl.when(rb == 0)
    def _init():
        out_ref[...] = jnp.zeros_like(out_ref)

    out_ref[...] += jnp.broadcast_to(contrib, out_ref.shape)


def kernel(feat_trainable, feat_criterion, dec_masks):
    ft = feat_trainable.reshape(_G, _GN, _C)
    fc = feat_criterion.reshape(_G, _GN, _C)
    dm = dec_masks.reshape(_G, 2, 16, 1024)

    out = pl.pallas_call(
        _fused_body,
        grid=(_G, _NRB),
        in_specs=[
            pl.BlockSpec((1, _GN, _C), lambda gi, rb: (gi, 0, 0)),
            pl.BlockSpec((1, _GN, _C), lambda gi, rb: (gi, 0, 0)),
            pl.BlockSpec((1, 2, 16, 1024), lambda gi, rb: (gi, 0, 0, 0)),
        ],
        out_specs=pl.BlockSpec((1, 1, 128), lambda gi, rb: (gi, 0, 0)),
        out_shape=jax.ShapeDtypeStruct((_G, 1, 128), jnp.float32),
        compiler_params=pltpu.CompilerParams(
            dimension_semantics=("parallel", "arbitrary")),
        scratch_shapes=[
            pltpu.VMEM((_GN, _C), jnp.float32),
            pltpu.VMEM((_GN, _C), jnp.float32),
            pltpu.VMEM((1, _GN), jnp.float32),
            pltpu.VMEM((_RB, 128), jnp.float32),
        ],
    )(fc, ft, dm)
    return jnp.sum(out[:, 0, 0])


# speedup vs baseline: 2.5773x; 1.1869x over previous
import jax
import jax.numpy as jnp
from jax.experimental import pallas as pl
from jax.experimental.pallas import tpu as pltpu

_TEMP = 0.1
_BASE_TEMP = 0.07
_TOPK = 16
_G = 8
_GN = 2048
_C = 128
_RB = 256
_NRB = _GN // _RB
_SCALE = -(_TEMP / _BASE_TEMP) / (_G * _GN * 2.0)


def _fused_body(fc_ref, ft_ref, dm_ref, out_ref, fcn, ftn, lab, ps_s, pc_s):
    gi = pl.program_id(0)
    rb = pl.program_id(1)

    @pl.when(rb == 0)
    def _prep():
        fc = fc_ref[0]
        nc = jnp.sqrt(jnp.sum(fc * fc, axis=-1, keepdims=True))
        fcn[...] = fc / jnp.maximum(nc, 1e-12)
        ft = ft_ref[0]
        nt = jnp.sqrt(jnp.sum(ft * ft, axis=-1, keepdims=True))
        ftn[...] = ft / jnp.maximum(nt, 1e-12)
        dm = dm_ref[0]
        mx = jnp.max(dm, axis=1, keepdims=True)
        sidx = jax.lax.broadcasted_iota(jnp.int32, dm.shape, 1)
        cand = jnp.where(dm == mx, sidx, dm.shape[1])
        lab[...] = jnp.min(cand, axis=1).reshape(1, _GN).astype(jnp.float32)

    r0 = rb * _RB
    fcn_all = fcn[...]
    ftn_all = ftn[...]
    fcb = fcn[pl.ds(r0, _RB), :]
    sim = jax.lax.dot_general(
        fcb, fcn_all, (((1,), (1,)), ((), ())),
        preferred_element_type=jnp.float32)

    neg = jnp.float32(-jnp.inf)
    m1 = jnp.full((_RB, 128), neg, jnp.float32)
    m2 = jnp.full((_RB, 128), neg, jnp.float32)
    m3 = jnp.full((_RB, 128), neg, jnp.float32)
    m4 = jnp.full((_RB, 128), neg, jnp.float32)
    for j in range(_GN // 128):
        v = sim[:, j * 128:(j + 1) * 128]
        hi = jnp.maximum(m1, v); v = jnp.minimum(m1, v); m1 = hi
        hi = jnp.maximum(m2, v); v = jnp.minimum(m2, v); m2 = hi
        hi = jnp.maximum(m3, v); v = jnp.minimum(m3, v); m3 = hi
        m4 = jnp.maximum(m4, v)
    cand = jnp.concatenate([m1, m2, m3, m4], axis=1)
    t = jnp.max(m1, axis=1, keepdims=True)
    for _ in range(_TOPK - 1):
        t = jnp.max(jnp.where(cand < t, cand, neg), axis=1, keepdims=True)
    ftb = ftn[pl.ds(r0, _RB), :]
    logits = jax.lax.dot_general(
        ftb, ftn_all, (((1,), (1,)), ((), ())),
        preferred_element_type=jnp.float32) / _TEMP

    dg_t = jnp.sum(ftb * ftb, axis=1, keepdims=True) / _TEMP
    dg_c = jnp.sum(fcb * fcb, axis=1, keepdims=True)
    denom = jnp.sum(jnp.exp(logits), axis=1, keepdims=True) - jnp.exp(dg_t)
    logden = jnp.log(denom)

    ge = sim >= t
    cnt_ge = jnp.sum(jnp.where(ge, 1.0, 0.0), axis=1, keepdims=True)
    sum_ge = jnp.sum(jnp.where(ge, logits, 0.0), axis=1, keepdims=True)
    diag_in = jnp.where(dg_c >= t, 1.0, 0.0)
    ps_s[...] = jnp.broadcast_to(sum_ge - diag_in * dg_t, (_RB, 128))
    pc_s[...] = jnp.broadcast_to(cnt_ge - diag_in, (_RB, 128))
    all_ok = jnp.all(cnt_ge == float(_TOPK))

    @pl.when(jnp.logical_not(all_ok))
    def _exact_topk():
        tt = jnp.full((_RB, 1), jnp.inf, jnp.float32)
        th = jnp.zeros((_RB, 1), jnp.float32)
        done = jnp.zeros((_RB, 1), jnp.bool_)
        for _ in range(_TOPK):
            m = jnp.max(jnp.where(sim < tt, sim, neg), axis=1, keepdims=True)
            c = jnp.sum(jnp.where(sim >= m, 1.0, 0.0), axis=1, keepdims=True)
            reach = c >= _TOPK
            hit = jnp.logical_and(jnp.logical_not(done), reach)
            th = jnp.where(hit, m, th)
            done = jnp.logical_or(done, reach)
            tt = jnp.where(done, tt, m)
        ge2 = sim >= th
        cnt2 = jnp.sum(jnp.where(ge2, 1.0, 0.0), axis=1, keepdims=True)
        sum2 = jnp.sum(jnp.where(ge2, logits, 0.0), axis=1, keepdims=True)
        dia2 = jnp.where(dg_c >= th, 1.0, 0.0)
        ps_s[...] = jnp.broadcast_to(sum2 - dia2 * dg_t, (_RB, 128))
        pc_s[...] = jnp.broadcast_to(cnt2 - dia2, (_RB, 128))

    pos_sum = ps_s[:, 0:1]
    pos_cnt = pc_s[:, 0:1]

    labr = lab[:, pl.ds(r0, _RB)].reshape(_RB, 1)
    labc = lab[...]
    leq = labr == labc
    semi_sum = jnp.sum(jnp.where(leq, logits, 0.0), axis=1,
                       keepdims=True) - dg_t
    semi_cnt = jnp.sum(jnp.where(leq, 1.0, 0.0), axis=1,
                       keepdims=True) - 1.0

    row_loss = ((pos_sum - pos_cnt * logden) / (pos_cnt + 1e-8)
                + (semi_sum - semi_cnt * logden) / (semi_cnt + 1e-8))
    contrib = (jnp.sum(row_loss) * _SCALE).reshape(1, 1)

    @pl.when(rb == 0)
    def _init():
        out_ref[...] = jnp.zeros_like(out_ref)

    out_ref[...] += jnp.broadcast_to(contrib, out_ref.shape)


def kernel(feat_trainable, feat_criterion, dec_masks):
    ft = feat_trainable.reshape(_G, _GN, _C)
    fc = feat_criterion.reshape(_G, _GN, _C)
    dm = dec_masks.reshape(_G, 2, 16, 1024)

    out = pl.pallas_call(
        _fused_body,
        grid=(_G, _NRB),
        in_specs=[
            pl.BlockSpec((1, _GN, _C), lambda gi, rb: (gi, 0, 0)),
            pl.BlockSpec((1, _GN, _C), lambda gi, rb: (gi, 0, 0)),
            pl.BlockSpec((1, 2, 16, 1024), lambda gi, rb: (gi, 0, 0, 0)),
        ],
        out_specs=pl.BlockSpec((1, 1, 128), lambda gi, rb: (gi, 0, 0)),
        out_shape=jax.ShapeDtypeStruct((_G, 1, 128), jnp.float32),
        compiler_params=pltpu.CompilerParams(
            dimension_semantics=("parallel", "arbitrary")),
        scratch_shapes=[
            pltpu.VMEM((_GN, _C), jnp.float32),
            pltpu.VMEM((_GN, _C), jnp.float32),
            pltpu.VMEM((1, _GN), jnp.float32),
            pltpu.VMEM((_RB, 128), jnp.float32),
            pltpu.VMEM((_RB, 128), jnp.float32),
        ],
    )(fc, ft, dm)
    return jnp.sum(out[:, 0, 0])


# speedup vs baseline: 2.5880x; 1.0042x over previous
import jax
import jax.numpy as jnp
from jax.experimental import pallas as pl
from jax.experimental.pallas import tpu as pltpu

_TEMP = 0.1
_BASE_TEMP = 0.07
_TOPK = 16
_G = 8
_GN = 2048
_C = 128
_RB = 256
_NRB = _GN // _RB
_SCALE = -(_TEMP / _BASE_TEMP) / (_G * _GN * 2.0)


def _fused_body(fc_ref, ft_ref, dm_ref, out_ref, fcn, ftn, lab, ps_s, pc_s):
    gi = pl.program_id(0)
    rb = pl.program_id(1)

    @pl.when(rb == 0)
    def _prep():
        fc = fc_ref[0]
        nc = jnp.sqrt(jnp.sum(fc * fc, axis=-1, keepdims=True))
        fcn[...] = fc / jnp.maximum(nc, 1e-12)
        ft = ft_ref[0]
        nt = jnp.sqrt(jnp.sum(ft * ft, axis=-1, keepdims=True))
        ftn[...] = ft / jnp.maximum(nt, 1e-12)
        dm = dm_ref[0]
        mx = jnp.max(dm, axis=1, keepdims=True)
        sidx = jax.lax.broadcasted_iota(jnp.int32, dm.shape, 1)
        cand = jnp.where(dm == mx, sidx, dm.shape[1])
        lab[...] = jnp.min(cand, axis=1).reshape(1, _GN).astype(jnp.float32)

    r0 = rb * _RB
    fcn_all = fcn[...]
    ftn_all = ftn[...]
    fcb = fcn[pl.ds(r0, _RB), :]
    sim = jax.lax.dot_general(
        fcb, fcn_all, (((1,), (1,)), ((), ())),
        preferred_element_type=jnp.float32)

    ftb = ftn[pl.ds(r0, _RB), :]
    logits = jax.lax.dot_general(
        ftb, ftn_all, (((1,), (1,)), ((), ())),
        preferred_element_type=jnp.float32) / _TEMP

    dg_t = jnp.sum(ftb * ftb, axis=1, keepdims=True) / _TEMP
    dg_c = jnp.sum(fcb * fcb, axis=1, keepdims=True)
    denom = jnp.sum(jnp.exp(logits), axis=1, keepdims=True) - jnp.exp(dg_t)
    logden = jnp.log(denom)

    labr = lab[:, pl.ds(r0, _RB)].reshape(_RB, 1)
    labc = lab[...]
    leq = labr == labc
    semi_sum = jnp.sum(jnp.where(leq, logits, 0.0), axis=1,
                       keepdims=True) - dg_t
    semi_cnt = jnp.sum(jnp.where(leq, 1.0, 0.0), axis=1,
                       keepdims=True) - 1.0

    neg = jnp.float32(-jnp.inf)
    m1 = jnp.full((_RB, 128), neg, jnp.float32)
    m2 = jnp.full((_RB, 128), neg, jnp.float32)
    m3 = jnp.full((_RB, 128), neg, jnp.float32)
    m4 = jnp.full((_RB, 128), neg, jnp.float32)
    for j in range(_GN // 128):
        v = sim[:, j * 128:(j + 1) * 128]
        hi = jnp.maximum(m1, v); v = jnp.minimum(m1, v); m1 = hi
        hi = jnp.maximum(m2, v); v = jnp.minimum(m2, v); m2 = hi
        hi = jnp.maximum(m3, v); v = jnp.minimum(m3, v); m3 = hi
        m4 = jnp.maximum(m4, v)
    cand = jnp.concatenate([m1, m2, m3, m4], axis=1)
    t = jnp.max(m1, axis=1, keepdims=True)
    for _ in range(_TOPK - 1):
        t = jnp.max(jnp.where(cand < t, cand, neg), axis=1, keepdims=True)
    ge = sim >= t
    cnt_ge = jnp.sum(jnp.where(ge, 1.0, 0.0), axis=1, keepdims=True)
    sum_ge = jnp.sum(jnp.where(ge, logits, 0.0), axis=1, keepdims=True)
    diag_in = jnp.where(dg_c >= t, 1.0, 0.0)
    ps_s[...] = jnp.broadcast_to(sum_ge - diag_in * dg_t, (_RB, 128))
    pc_s[...] = jnp.broadcast_to(cnt_ge - diag_in, (_RB, 128))
    all_ok = jnp.all(cnt_ge == float(_TOPK))

    @pl.when(jnp.logical_not(all_ok))
    def _exact_topk():
        tt = jnp.full((_RB, 1), jnp.inf, jnp.float32)
        th = jnp.zeros((_RB, 1), jnp.float32)
        done = jnp.zeros((_RB, 1), jnp.bool_)
        for _ in range(_TOPK):
            m = jnp.max(jnp.where(sim < tt, sim, neg), axis=1, keepdims=True)
            c = jnp.sum(jnp.where(sim >= m, 1.0, 0.0), axis=1, keepdims=True)
            reach = c >= _TOPK
            hit = jnp.logical_and(jnp.logical_not(done), reach)
            th = jnp.where(hit, m, th)
            done = jnp.logical_or(done, reach)
            tt = jnp.where(done, tt, m)
        ge2 = sim >= th
        cnt2 = jnp.sum(jnp.where(ge2, 1.0, 0.0), axis=1, keepdims=True)
        sum2 = jnp.sum(jnp.where(ge2, logits, 0.0), axis=1, keepdims=True)
        dia2 = jnp.where(dg_c >= th, 1.0, 0.0)
        ps_s[...] = jnp.broadcast_to(sum2 - dia2 * dg_t, (_RB, 128))
        pc_s[...] = jnp.broadcast_to(cnt2 - dia2, (_RB, 128))

    pos_sum = ps_s[:, 0:1]
    pos_cnt = pc_s[:, 0:1]

    row_loss = ((pos_sum - pos_cnt * logden) / (pos_cnt + 1e-8)
                + (semi_sum - semi_cnt * logden) / (semi_cnt + 1e-8))
    contrib = (jnp.sum(row_loss) * _SCALE).reshape(1, 1)

    @pl.when(rb == 0)
    def _init():
        out_ref[...] = jnp.zeros_like(out_ref)

    out_ref[...] += jnp.broadcast_to(contrib, out_ref.shape)


def kernel(feat_trainable, feat_criterion, dec_masks):
    ft = feat_trainable.reshape(_G, _GN, _C)
    fc = feat_criterion.reshape(_G, _GN, _C)
    dm = dec_masks.reshape(_G, 2, 16, 1024)

    out = pl.pallas_call(
        _fused_body,
        grid=(_G, _NRB),
        in_specs=[
            pl.BlockSpec((1, _GN, _C), lambda gi, rb: (gi, 0, 0)),
            pl.BlockSpec((1, _GN, _C), lambda gi, rb: (gi, 0, 0)),
            pl.BlockSpec((1, 2, 16, 1024), lambda gi, rb: (gi, 0, 0, 0)),
        ],
        out_specs=pl.BlockSpec((1, 1, 128), lambda gi, rb: (gi, 0, 0)),
        out_shape=jax.ShapeDtypeStruct((_G, 1, 128), jnp.float32),
        compiler_params=pltpu.CompilerParams(
            dimension_semantics=("parallel", "arbitrary")),
        scratch_shapes=[
            pltpu.VMEM((_GN, _C), jnp.float32),
            pltpu.VMEM((_GN, _C), jnp.float32),
            pltpu.VMEM((1, _GN), jnp.float32),
            pltpu.VMEM((_RB, 128), jnp.float32),
            pltpu.VMEM((_RB, 128), jnp.float32),
        ],
    )(fc, ft, dm)
    return jnp.sum(out[:, 0, 0])


# speedup vs baseline: 2.8658x; 1.1073x over previous
import jax
import jax.numpy as jnp
from jax.experimental import pallas as pl
from jax.experimental.pallas import tpu as pltpu

_TEMP = 0.1
_BASE_TEMP = 0.07
_TOPK = 16
_G = 8
_GN = 2048
_C = 128
_RB = 256
_NRB = _GN // _RB
_SCALE = -(_TEMP / _BASE_TEMP) / (_G * _GN * 2.0)


def _fused_body(fc_ref, ft_ref, dm_ref, out_ref, fcn, ftn, lab, oh, cnts,
                ps_s, pc_s):
    gi = pl.program_id(0)
    rb = pl.program_id(1)

    @pl.when(rb == 0)
    def _prep():
        fc = fc_ref[0]
        nc = jnp.sqrt(jnp.sum(fc * fc, axis=-1, keepdims=True))
        fcn[...] = fc / jnp.maximum(nc, 1e-12)
        ft = ft_ref[0]
        nt = jnp.sqrt(jnp.sum(ft * ft, axis=-1, keepdims=True))
        ftn[...] = ft / jnp.maximum(nt, 1e-12)
        dm = dm_ref[0]
        mx = jnp.max(dm, axis=1, keepdims=True)
        sidx = jax.lax.broadcasted_iota(jnp.int32, dm.shape, 1)
        cand = jnp.where(dm == mx, sidx, dm.shape[1])
        lb = jnp.min(cand, axis=1).reshape(1, _GN).astype(jnp.float32)
        lab[...] = lb
        cidx = jax.lax.broadcasted_iota(jnp.int32, (_GN, 128), 1)
        ohv = jnp.where(lb.reshape(_GN, 1) == cidx.astype(jnp.float32),
                        1.0, 0.0)
        oh[...] = ohv
        cnts[...] = jnp.broadcast_to(
            jnp.sum(ohv, axis=0, keepdims=True), (8, 128))

    r0 = rb * _RB
    fcn_all = fcn[...]
    ftn_all = ftn[...]
    fcb = fcn[pl.ds(r0, _RB), :]
    sim = jax.lax.dot_general(
        fcb, fcn_all, (((1,), (1,)), ((), ())),
        preferred_element_type=jnp.float32)

    ftb = ftn[pl.ds(r0, _RB), :]
    logits = jax.lax.dot_general(
        ftb, ftn_all, (((1,), (1,)), ((), ())),
        preferred_element_type=jnp.float32) / _TEMP

    dg_t = jnp.sum(ftb * ftb, axis=1, keepdims=True) / _TEMP
    dg_c = jnp.sum(fcb * fcb, axis=1, keepdims=True)
    denom = jnp.sum(jnp.exp(logits), axis=1, keepdims=True) - jnp.exp(dg_t)
    logden = jnp.log(denom)

    slab = jax.lax.dot_general(
        logits, oh[...], (((1,), (0,)), ((), ())),
        preferred_element_type=jnp.float32)
    labr = lab[:, pl.ds(r0, _RB)].reshape(_RB, 1)
    ridx = jax.lax.broadcasted_iota(jnp.int32, (_RB, 128), 1)
    ohr = jnp.where(labr == ridx.astype(jnp.float32), 1.0, 0.0)
    semi_sum = jnp.sum(ohr * slab, axis=1, keepdims=True) - dg_t
    semi_cnt = jnp.sum(ohr * cnts[0:1, :], axis=1, keepdims=True) - 1.0

    neg = jnp.float32(-jnp.inf)
    m1 = jnp.full((_RB, 128), neg, jnp.float32)
    m2 = jnp.full((_RB, 128), neg, jnp.float32)
    m3 = jnp.full((_RB, 128), neg, jnp.float32)
    m4 = jnp.full((_RB, 128), neg, jnp.float32)
    for j in range(_GN // 128):
        v = sim[:, j * 128:(j + 1) * 128]
        hi = jnp.maximum(m1, v); v = jnp.minimum(m1, v); m1 = hi
        hi = jnp.maximum(m2, v); v = jnp.minimum(m2, v); m2 = hi
        hi = jnp.maximum(m3, v); v = jnp.minimum(m3, v); m3 = hi
        m4 = jnp.maximum(m4, v)
    cand = jnp.concatenate([m1, m2, m3, m4], axis=1)
    t = jnp.max(m1, axis=1, keepdims=True)
    for _ in range(_TOPK - 1):
        t = jnp.max(jnp.where(cand < t, cand, neg), axis=1, keepdims=True)
    ge = sim >= t
    cnt_ge = jnp.sum(jnp.where(ge, 1.0, 0.0), axis=1, keepdims=True)
    sum_ge = jnp.sum(jnp.where(ge, logits, 0.0), axis=1, keepdims=True)
    diag_in = jnp.where(dg_c >= t, 1.0, 0.0)
    ps_s[...] = jnp.broadcast_to(sum_ge - diag_in * dg_t, (_RB, 128))
    pc_s[...] = jnp.broadcast_to(cnt_ge - diag_in, (_RB, 128))
    all_ok = jnp.all(cnt_ge == float(_TOPK))

    @pl.when(jnp.logical_not(all_ok))
    def _exact_topk():
        tt = jnp.full((_RB, 1), jnp.inf, jnp.float32)
        th = jnp.zeros((_RB, 1), jnp.float32)
        done = jnp.zeros((_RB, 1), jnp.bool_)
        for _ in range(_TOPK):
            m = jnp.max(jnp.where(sim < tt, sim, neg), axis=1, keepdims=True)
            c = jnp.sum(jnp.where(sim >= m, 1.0, 0.0), axis=1, keepdims=True)
            reach = c >= _TOPK
            hit = jnp.logical_and(jnp.logical_not(done), reach)
            th = jnp.where(hit, m, th)
            done = jnp.logical_or(done, reach)
            tt = jnp.where(done, tt, m)
        ge2 = sim >= th
        cnt2 = jnp.sum(jnp.where(ge2, 1.0, 0.0), axis=1, keepdims=True)
        sum2 = jnp.sum(jnp.where(ge2, logits, 0.0), axis=1, keepdims=True)
        dia2 = jnp.where(dg_c >= th, 1.0, 0.0)
        ps_s[...] = jnp.broadcast_to(sum2 - dia2 * dg_t, (_RB, 128))
        pc_s[...] = jnp.broadcast_to(cnt2 - dia2, (_RB, 128))

    pos_sum = ps_s[:, 0:1]
    pos_cnt = pc_s[:, 0:1]

    row_loss = ((pos_sum - pos_cnt * logden) / (pos_cnt + 1e-8)
                + (semi_sum - semi_cnt * logden) / (semi_cnt + 1e-8))
    contrib = (jnp.sum(row_loss) * _SCALE).reshape(1, 1)

    @pl.when(rb == 0)
    def _init():
        out_ref[...] = jnp.zeros_like(out_ref)

    out_ref[...] += jnp.broadcast_to(contrib, out_ref.shape)


def kernel(feat_trainable, feat_criterion, dec_masks):
    ft = feat_trainable.reshape(_G, _GN, _C)
    fc = feat_criterion.reshape(_G, _GN, _C)
    dm = dec_masks.reshape(_G, 2, 16, 1024)

    out = pl.pallas_call(
        _fused_body,
        grid=(_G, _NRB),
        in_specs=[
            pl.BlockSpec((1, _GN, _C), lambda gi, rb: (gi, 0, 0)),
            pl.BlockSpec((1, _GN, _C), lambda gi, rb: (gi, 0, 0)),
            pl.BlockSpec((1, 2, 16, 1024), lambda gi, rb: (gi, 0, 0, 0)),
        ],
        out_specs=pl.BlockSpec((1, 1, 128), lambda gi, rb: (gi, 0, 0)),
        out_shape=jax.ShapeDtypeStruct((_G, 1, 128), jnp.float32),
        compiler_params=pltpu.CompilerParams(
            dimension_semantics=("parallel", "arbitrary")),
        scratch_shapes=[
            pltpu.VMEM((_GN, _C), jnp.float32),
            pltpu.VMEM((_GN, _C), jnp.float32),
            pltpu.VMEM((1, _GN), jnp.float32),
            pltpu.VMEM((_GN, 128), jnp.float32),
            pltpu.VMEM((8, 128), jnp.float32),
            pltpu.VMEM((_RB, 128), jnp.float32),
            pltpu.VMEM((_RB, 128), jnp.float32),
        ],
    )(fc, ft, dm)
    return jnp.sum(out[:, 0, 0])


# speedup vs baseline: 3.4165x; 1.1922x over previous
import jax
import jax.numpy as jnp
from jax.experimental import pallas as pl
from jax.experimental.pallas import tpu as pltpu

_TEMP = 0.1
_BASE_TEMP = 0.07
_TOPK = 16
_G = 8
_GN = 2048
_C = 128
_RB = 512
_NRB = _GN // _RB
_SCALE = -(_TEMP / _BASE_TEMP) / (_G * _GN * 2.0)


def _fused_body(fc_ref, ft_ref, dm_ref, out_ref, fcn, ftn, lab, oh, cnts,
                ps_s, pc_s):
    gi = pl.program_id(0)
    rb = pl.program_id(1)

    @pl.when(rb == 0)
    def _prep():
        fc = fc_ref[0]
        nc = jnp.sqrt(jnp.sum(fc * fc, axis=-1, keepdims=True))
        fcn[...] = fc / jnp.maximum(nc, 1e-12)
        ft = ft_ref[0]
        nt = jnp.sqrt(jnp.sum(ft * ft, axis=-1, keepdims=True))
        ftn[...] = ft / jnp.maximum(nt, 1e-12)
        dm = dm_ref[0]
        mx = jnp.max(dm, axis=1, keepdims=True)
        sidx = jax.lax.broadcasted_iota(jnp.int32, dm.shape, 1)
        cand = jnp.where(dm == mx, sidx, dm.shape[1])
        lb = jnp.min(cand, axis=1).reshape(1, _GN).astype(jnp.float32)
        lab[...] = lb
        cidx = jax.lax.broadcasted_iota(jnp.int32, (_GN, 128), 1)
        ohv = jnp.where(lb.reshape(_GN, 1) == cidx.astype(jnp.float32),
                        1.0, 0.0)
        oh[...] = ohv
        cnts[...] = jnp.broadcast_to(
            jnp.sum(ohv, axis=0, keepdims=True), (8, 128))

    r0 = rb * _RB
    fcn_all = fcn[...]
    ftn_all = ftn[...]
    fcb = fcn[pl.ds(r0, _RB), :]
    sim = jax.lax.dot_general(
        fcb, fcn_all, (((1,), (1,)), ((), ())),
        preferred_element_type=jnp.float32)

    ftb = ftn[pl.ds(r0, _RB), :]
    logits = jax.lax.dot_general(
        ftb, ftn_all, (((1,), (1,)), ((), ())),
        preferred_element_type=jnp.float32) / _TEMP

    dg_t = jnp.sum(ftb * ftb, axis=1, keepdims=True) / _TEMP
    dg_c = jnp.sum(fcb * fcb, axis=1, keepdims=True)
    denom = jnp.sum(jnp.exp(logits), axis=1, keepdims=True) - jnp.exp(dg_t)
    logden = jnp.log(denom)

    slab = jax.lax.dot_general(
        logits, oh[...], (((1,), (0,)), ((), ())),
        preferred_element_type=jnp.float32)
    labr = lab[:, pl.ds(r0, _RB)].reshape(_RB, 1)
    ridx = jax.lax.broadcasted_iota(jnp.int32, (_RB, 128), 1)
    ohr = jnp.where(labr == ridx.astype(jnp.float32), 1.0, 0.0)
    semi_sum = jnp.sum(ohr * slab, axis=1, keepdims=True) - dg_t
    semi_cnt = jnp.sum(ohr * cnts[0:1, :], axis=1, keepdims=True) - 1.0

    neg = jnp.float32(-jnp.inf)
    m1 = jnp.full((_RB, 128), neg, jnp.float32)
    m2 = jnp.full((_RB, 128), neg, jnp.float32)
    m3 = jnp.full((_RB, 128), neg, jnp.float32)
    m4 = jnp.full((_RB, 128), neg, jnp.float32)
    for j in range(_GN // 128):
        v = sim[:, j * 128:(j + 1) * 128]
        hi = jnp.maximum(m1, v); v = jnp.minimum(m1, v); m1 = hi
        hi = jnp.maximum(m2, v); v = jnp.minimum(m2, v); m2 = hi
        hi = jnp.maximum(m3, v); v = jnp.minimum(m3, v); m3 = hi
        m4 = jnp.maximum(m4, v)
    cand = jnp.concatenate([m1, m2, m3, m4], axis=1)
    t = jnp.max(m1, axis=1, keepdims=True)
    for _ in range(_TOPK - 1):
        t = jnp.max(jnp.where(cand < t, cand, neg), axis=1, keepdims=True)
    ge = sim >= t
    cnt_ge = jnp.sum(jnp.where(ge, 1.0, 0.0), axis=1, keepdims=True)
    sum_ge = jnp.sum(jnp.where(ge, logits, 0.0), axis=1, keepdims=True)
    diag_in = jnp.where(dg_c >= t, 1.0, 0.0)
    ps_s[...] = jnp.broadcast_to(sum_ge - diag_in * dg_t, (_RB, 128))
    pc_s[...] = jnp.broadcast_to(cnt_ge - diag_in, (_RB, 128))
    all_ok = jnp.all(cnt_ge == float(_TOPK))

    @pl.when(jnp.logical_not(all_ok))
    def _exact_topk():
        tt = jnp.full((_RB, 1), jnp.inf, jnp.float32)
        th = jnp.zeros((_RB, 1), jnp.float32)
        done = jnp.zeros((_RB, 1), jnp.bool_)
        for _ in range(_TOPK):
            m = jnp.max(jnp.where(sim < tt, sim, neg), axis=1, keepdims=True)
            c = jnp.sum(jnp.where(sim >= m, 1.0, 0.0), axis=1, keepdims=True)
            reach = c >= _TOPK
            hit = jnp.logical_and(jnp.logical_not(done), reach)
            th = jnp.where(hit, m, th)
            done = jnp.logical_or(done, reach)
            tt = jnp.where(done, tt, m)
        ge2 = sim >= th
        cnt2 = jnp.sum(jnp.where(ge2, 1.0, 0.0), axis=1, keepdims=True)
        sum2 = jnp.sum(jnp.where(ge2, logits, 0.0), axis=1, keepdims=True)
        dia2 = jnp.where(dg_c >= th, 1.0, 0.0)
        ps_s[...] = jnp.broadcast_to(sum2 - dia2 * dg_t, (_RB, 128))
        pc_s[...] = jnp.broadcast_to(cnt2 - dia2, (_RB, 128))

    pos_sum = ps_s[:, 0:1]
    pos_cnt = pc_s[:, 0:1]

    row_loss = ((pos_sum - pos_cnt * logden) / (pos_cnt + 1e-8)
                + (semi_sum - semi_cnt * logden) / (semi_cnt + 1e-8))
    contrib = (jnp.sum(row_loss) * _SCALE).reshape(1, 1)

    @pl.when(rb == 0)
    def _init():
        out_ref[...] = jnp.zeros_like(out_ref)

    out_ref[...] += jnp.broadcast_to(contrib, out_ref.shape)


def kernel(feat_trainable, feat_criterion, dec_masks):
    ft = feat_trainable.reshape(_G, _GN, _C)
    fc = feat_criterion.reshape(_G, _GN, _C)
    dm = dec_masks.reshape(_G, 2, 16, 1024)

    out = pl.pallas_call(
        _fused_body,
        grid=(_G, _NRB),
        in_specs=[
            pl.BlockSpec((1, _GN, _C), lambda gi, rb: (gi, 0, 0)),
            pl.BlockSpec((1, _GN, _C), lambda gi, rb: (gi, 0, 0)),
            pl.BlockSpec((1, 2, 16, 1024), lambda gi, rb: (gi, 0, 0, 0)),
        ],
        out_specs=pl.BlockSpec((1, 1, 128), lambda gi, rb: (gi, 0, 0)),
        out_shape=jax.ShapeDtypeStruct((_G, 1, 128), jnp.float32),
        compiler_params=pltpu.CompilerParams(
            dimension_semantics=("parallel", "arbitrary")),
        scratch_shapes=[
            pltpu.VMEM((_GN, _C), jnp.float32),
            pltpu.VMEM((_GN, _C), jnp.float32),
            pltpu.VMEM((1, _GN), jnp.float32),
            pltpu.VMEM((_GN, 128), jnp.float32),
            pltpu.VMEM((8, 128), jnp.float32),
            pltpu.VMEM((_RB, 128), jnp.float32),
            pltpu.VMEM((_RB, 128), jnp.float32),
        ],
    )(fc, ft, dm)
    return jnp.sum(out[:, 0, 0])


# speedup vs baseline: 3.4272x; 1.0031x over previous
import jax
import jax.numpy as jnp
from jax.experimental import pallas as pl
from jax.experimental.pallas import tpu as pltpu

_TEMP = 0.1
_BASE_TEMP = 0.07
_TOPK = 16
_G = 8
_GN = 2048
_C = 128
_RB = 1024
_NRB = _GN // _RB
_SCALE = -(_TEMP / _BASE_TEMP) / (_G * _GN * 2.0)


def _fused_body(fc_ref, ft_ref, dm_ref, out_ref, fcn, ftn, lab, oh, cnts,
                ps_s, pc_s):
    gi = pl.program_id(0)
    rb = pl.program_id(1)

    @pl.when(rb == 0)
    def _prep():
        fc = fc_ref[0]
        nc = jnp.sqrt(jnp.sum(fc * fc, axis=-1, keepdims=True))
        fcn[...] = fc / jnp.maximum(nc, 1e-12)
        ft = ft_ref[0]
        nt = jnp.sqrt(jnp.sum(ft * ft, axis=-1, keepdims=True))
        ftn[...] = ft / jnp.maximum(nt, 1e-12)
        dm = dm_ref[0]
        mx = jnp.max(dm, axis=1, keepdims=True)
        sidx = jax.lax.broadcasted_iota(jnp.int32, dm.shape, 1)
        cand = jnp.where(dm == mx, sidx, dm.shape[1])
        lb = jnp.min(cand, axis=1).reshape(1, _GN).astype(jnp.float32)
        lab[...] = lb
        cidx = jax.lax.broadcasted_iota(jnp.int32, (_GN, 128), 1)
        ohv = jnp.where(lb.reshape(_GN, 1) == cidx.astype(jnp.float32),
                        1.0, 0.0)
        oh[...] = ohv
        cnts[...] = jnp.broadcast_to(
            jnp.sum(ohv, axis=0, keepdims=True), (8, 128))

    r0 = rb * _RB
    fcn_all = fcn[...]
    ftn_all = ftn[...]
    fcb = fcn[pl.ds(r0, _RB), :]
    sim = jax.lax.dot_general(
        fcb, fcn_all, (((1,), (1,)), ((), ())),
        preferred_element_type=jnp.float32)

    ftb = ftn[pl.ds(r0, _RB), :]
    logits = jax.lax.dot_general(
        ftb, ftn_all, (((1,), (1,)), ((), ())),
        preferred_element_type=jnp.float32) / _TEMP

    dg_t = jnp.sum(ftb * ftb, axis=1, keepdims=True) / _TEMP
    dg_c = jnp.sum(fcb * fcb, axis=1, keepdims=True)
    denom = jnp.sum(jnp.exp(logits), axis=1, keepdims=True) - jnp.exp(dg_t)
    logden = jnp.log(denom)

    slab = jax.lax.dot_general(
        logits, oh[...], (((1,), (0,)), ((), ())),
        preferred_element_type=jnp.float32)
    labr = lab[:, pl.ds(r0, _RB)].reshape(_RB, 1)
    ridx = jax.lax.broadcasted_iota(jnp.int32, (_RB, 128), 1)
    ohr = jnp.where(labr == ridx.astype(jnp.float32), 1.0, 0.0)
    semi_sum = jnp.sum(ohr * slab, axis=1, keepdims=True) - dg_t
    semi_cnt = jnp.sum(ohr * cnts[0:1, :], axis=1, keepdims=True) - 1.0

    neg = jnp.float32(-jnp.inf)
    m1 = jnp.full((_RB, 128), neg, jnp.float32)
    m2 = jnp.full((_RB, 128), neg, jnp.float32)
    m3 = jnp.full((_RB, 128), neg, jnp.float32)
    m4 = jnp.full((_RB, 128), neg, jnp.float32)
    for j in range(_GN // 128):
        v = sim[:, j * 128:(j + 1) * 128]
        hi = jnp.maximum(m1, v); v = jnp.minimum(m1, v); m1 = hi
        hi = jnp.maximum(m2, v); v = jnp.minimum(m2, v); m2 = hi
        hi = jnp.maximum(m3, v); v = jnp.minimum(m3, v); m3 = hi
        m4 = jnp.maximum(m4, v)
    cand = jnp.concatenate([m1, m2, m3, m4], axis=1)
    t = jnp.max(m1, axis=1, keepdims=True)
    for _ in range(_TOPK - 1):
        t = jnp.max(jnp.where(cand < t, cand, neg), axis=1, keepdims=True)
    ge = sim >= t
    cnt_ge = jnp.sum(jnp.where(ge, 1.0, 0.0), axis=1, keepdims=True)
    sum_ge = jnp.sum(jnp.where(ge, logits, 0.0), axis=1, keepdims=True)
    diag_in = jnp.where(dg_c >= t, 1.0, 0.0)
    ps_s[...] = jnp.broadcast_to(sum_ge - diag_in * dg_t, (_RB, 128))
    pc_s[...] = jnp.broadcast_to(cnt_ge - diag_in, (_RB, 128))
    all_ok = jnp.all(cnt_ge == float(_TOPK))

    @pl.when(jnp.logical_not(all_ok))
    def _exact_topk():
        tt = jnp.full((_RB, 1), jnp.inf, jnp.float32)
        th = jnp.zeros((_RB, 1), jnp.float32)
        done = jnp.zeros((_RB, 1), jnp.bool_)
        for _ in range(_TOPK):
            m = jnp.max(jnp.where(sim < tt, sim, neg), axis=1, keepdims=True)
            c = jnp.sum(jnp.where(sim >= m, 1.0, 0.0), axis=1, keepdims=True)
            reach = c >= _TOPK
            hit = jnp.logical_and(jnp.logical_not(done), reach)
            th = jnp.where(hit, m, th)
            done = jnp.logical_or(done, reach)
            tt = jnp.where(done, tt, m)
        ge2 = sim >= th
        cnt2 = jnp.sum(jnp.where(ge2, 1.0, 0.0), axis=1, keepdims=True)
        sum2 = jnp.sum(jnp.where(ge2, logits, 0.0), axis=1, keepdims=True)
        dia2 = jnp.where(dg_c >= th, 1.0, 0.0)
        ps_s[...] = jnp.broadcast_to(sum2 - dia2 * dg_t, (_RB, 128))
        pc_s[...] = jnp.broadcast_to(cnt2 - dia2, (_RB, 128))

    pos_sum = ps_s[:, 0:1]
    pos_cnt = pc_s[:, 0:1]

    row_loss = ((pos_sum - pos_cnt * logden) / (pos_cnt + 1e-8)
                + (semi_sum - semi_cnt * logden) / (semi_cnt + 1e-8))
    contrib = (jnp.sum(row_loss) * _SCALE).reshape(1, 1)

    @pl.when(rb == 0)
    def _init():
        out_ref[...] = jnp.zeros_like(out_ref)

    out_ref[...] += jnp.broadcast_to(contrib, out_ref.shape)


def kernel(feat_trainable, feat_criterion, dec_masks):
    ft = feat_trainable.reshape(_G, _GN, _C)
    fc = feat_criterion.reshape(_G, _GN, _C)
    dm = dec_masks.reshape(_G, 2, 16, 1024)

    out = pl.pallas_call(
        _fused_body,
        grid=(_G, _NRB),
        in_specs=[
            pl.BlockSpec((1, _GN, _C), lambda gi, rb: (gi, 0, 0)),
            pl.BlockSpec((1, _GN, _C), lambda gi, rb: (gi, 0, 0)),
            pl.BlockSpec((1, 2, 16, 1024), lambda gi, rb: (gi, 0, 0, 0)),
        ],
        out_specs=pl.BlockSpec((1, 1, 128), lambda gi, rb: (gi, 0, 0)),
        out_shape=jax.ShapeDtypeStruct((_G, 1, 128), jnp.float32),
        compiler_params=pltpu.CompilerParams(
            dimension_semantics=("parallel", "arbitrary")),
        scratch_shapes=[
            pltpu.VMEM((_GN, _C), jnp.float32),
            pltpu.VMEM((_GN, _C), jnp.float32),
            pltpu.VMEM((1, _GN), jnp.float32),
            pltpu.VMEM((_GN, 128), jnp.float32),
            pltpu.VMEM((8, 128), jnp.float32),
            pltpu.VMEM((_RB, 128), jnp.float32),
            pltpu.VMEM((_RB, 128), jnp.float32),
        ],
    )(fc, ft, dm)
    return jnp.sum(out[:, 0, 0])
